# baseline (speedup 1.0000x reference)
import jax
import jax.numpy as jnp
from jax import lax
from jax.experimental import pallas as pl
from jax.experimental.pallas import tpu as pltpu
from jax.experimental.pallas import tpu_sc as plsc

DEPTH = 1000
T = 16384

_INFO = plsc.get_sparse_core_info()
NC = _INFO.num_cores
NS = _INFO.num_subcores
L = _INFO.num_lanes
NW = NC * NS
TPW = T // NW
COLS = 128
NBC = TPW // COLS
RQ = (256, 248, 248, 248)
RLO = (0, 256, 504, 752)
NQ = len(RQ)
SUBS = [(c, q) for c in range(NBC) for q in range(NQ)]


def _body(x_hbm, out_hbm, idx_v, buf0, buf1, buf2, buf3,
          sem0, sem1, sem2, sem3, isem):
    wid = lax.axis_index("s") * NC + lax.axis_index("c")
    base = wid * TPW

    idx_cp = pltpu.make_async_copy(x_hbm.at[0, pl.ds(base, TPW)], idx_v, isem)
    idx_cp.start()

    bufs = (buf0, buf1, buf2, buf3)
    sems = (sem0, sem1, sem2, sem3)

    def _zero(buf, nrows):
        def body(i, carry):
            z = jnp.zeros((L,), jnp.float32)
            for r in range(8):
                for c in range(COLS // L):
                    buf[i * 8 + r, pl.ds(c * L, L)] = z
            return carry

        lax.fori_loop(0, nrows // 8, body, 0)

    _zero(buf0, RQ[0])
    idx_cp.wait()

    iota = lax.iota(jnp.int32, L)
    onesv = jnp.full((L,), 1.0, jnp.float32)
    zerov = jnp.zeros((L,), jnp.float32)

    def _scatter(buf, cblk, q, val):
        for c in range(COLS // L):
            colv = iota + c * L
            xv = idx_v[pl.ds(cblk * COLS + c * L, L)]
            xl = xv - RLO[q]
            mask = (xl >= 0) & (xl < RQ[q])
            plsc.store_scatter(buf, [xl, colv], val, mask=mask)

    copies = [None] * len(SUBS)
    for k, (cblk, q) in enumerate(SUBS):
        buf = bufs[q]
        sem = sems[q]
        if k >= NQ:
            copies[k - NQ].wait()
            pc, _ = SUBS[k - NQ]
            _scatter(buf, pc, q, zerov)
        _scatter(buf, cblk, q, onesv)
        dst = out_hbm.at[pl.ds(RLO[q], RQ[q]),
                         pl.ds(base + cblk * COLS, COLS)]
        copies[k] = pltpu.make_async_copy(buf, dst, sem)
        copies[k].start()
        if k < NQ - 1:
            _zero(bufs[k + 1], RQ[k + 1])
    for k in range(len(SUBS) - NQ, len(SUBS)):
        copies[k].wait()


@jax.jit
def _onehot_sc(x):
    k = pl.kernel(
        _body,
        out_type=jax.ShapeDtypeStruct((DEPTH, T), jnp.float32),
        mesh=plsc.VectorSubcoreMesh(core_axis_name="c", subcore_axis_name="s"),
        scratch_types=[
            pltpu.VMEM((TPW,), jnp.int32),
            pltpu.VMEM((RQ[0], COLS), jnp.float32),
            pltpu.VMEM((RQ[1], COLS), jnp.float32),
            pltpu.VMEM((RQ[2], COLS), jnp.float32),
            pltpu.VMEM((RQ[3], COLS), jnp.float32),
            pltpu.SemaphoreType.DMA,
            pltpu.SemaphoreType.DMA,
            pltpu.SemaphoreType.DMA,
            pltpu.SemaphoreType.DMA,
            pltpu.SemaphoreType.DMA,
        ],
        compiler_params=pltpu.CompilerParams(
            needs_layout_passes=False,
            disable_bounds_checks=True,
            disable_semaphore_checks=True,
            skip_device_barrier=True,
        ),
    )
    out_t = k(x)
    return out_t.T[None, :, :]


def kernel(x, ones):
    del ones
    return _onehot_sc(x)


# speedup vs baseline: 1.0202x; 1.0202x over previous
import jax
import jax.numpy as jnp
from jax import lax
from jax.experimental import pallas as pl
from jax.experimental.pallas import tpu as pltpu
from jax.experimental.pallas import tpu_sc as plsc

DEPTH = 1000
T = 16384

_INFO = plsc.get_sparse_core_info()
NC = _INFO.num_cores
NS = _INFO.num_subcores
L = _INFO.num_lanes
NW = NC * NS
TPW = T // NW
COLS = 128
NBC = TPW // COLS
R0 = 504
R1 = DEPTH - R0
SUBS = [(c, r) for c in range(NBC) for r in range(2)]


def _body(x_hbm, out_hbm, idx_v, buf0, buf1, sem0, sem1, isem):
    wid = lax.axis_index("s") * NC + lax.axis_index("c")
    base = wid * TPW

    idx_cp = pltpu.make_async_copy(x_hbm.at[0, pl.ds(base, TPW)], idx_v, isem)
    idx_cp.start()

    def _zero(buf, nrows):
        def body(i, carry):
            z = jnp.zeros((L,), jnp.float32)
            for r in range(8):
                for c in range(COLS // L):
                    buf[i * 8 + r, pl.ds(c * L, L)] = z
            return carry

        lax.fori_loop(0, nrows // 8, body, 0)

    _zero(buf0, R0)
    idx_cp.wait()

    iota = lax.iota(jnp.int32, L)
    onesv = jnp.full((L,), 1.0, jnp.float32)
    zerov = jnp.zeros((L,), jnp.float32)
    bufs = (buf0, buf1)
    sems = (sem0, sem1)

    def _scatter(buf, cblk, rhalf, val):
        lo = rhalf * R0
        nrows = R1 if rhalf else R0
        for c in range(COLS // L):
            colv = iota + c * L
            xv = idx_v[pl.ds(cblk * COLS + c * L, L)]
            xl = xv - lo
            mask = (xl >= 0) & (xl < nrows)
            plsc.store_scatter(buf, [xl, colv], val, mask=mask)

    copies = [None] * len(SUBS)
    for k, (cblk, rhalf) in enumerate(SUBS):
        buf = bufs[k % 2]
        sem = sems[k % 2]
        if k >= 2:
            copies[k - 2].wait()
            pc, pr = SUBS[k - 2]
            _scatter(buf, pc, pr, zerov)
        _scatter(buf, cblk, rhalf, onesv)
        nrows = R1 if rhalf else R0
        dst = out_hbm.at[pl.ds(rhalf * R0, nrows),
                         pl.ds(base + cblk * COLS, COLS)]
        copies[k] = pltpu.make_async_copy(buf.at[pl.ds(0, nrows), :], dst, sem)
        copies[k].start()
        if k == 0:
            _zero(buf1, R0)
    copies[len(SUBS) - 2].wait()
    copies[len(SUBS) - 1].wait()


@jax.jit
def _onehot_sc(x):
    k = pl.kernel(
        _body,
        out_type=jax.ShapeDtypeStruct((DEPTH, T), jnp.float32),
        mesh=plsc.VectorSubcoreMesh(core_axis_name="c", subcore_axis_name="s"),
        scratch_types=[
            pltpu.VMEM((TPW,), jnp.int32),
            pltpu.VMEM((R0, COLS), jnp.float32),
            pltpu.VMEM((R0, COLS), jnp.float32),
            pltpu.SemaphoreType.DMA,
            pltpu.SemaphoreType.DMA,
            pltpu.SemaphoreType.DMA,
        ],
        compiler_params=pltpu.CompilerParams(
            needs_layout_passes=False,
            disable_bounds_checks=True,
            disable_semaphore_checks=True,
            skip_device_barrier=True,
        ),
    )
    out_t = k(x)
    return out_t.T[None, :, :]


def kernel(x, ones):
    del ones
    return _onehot_sc(x)
